# Initial kernel scaffold; baseline (speedup 1.0000x reference)
import jax
import jax.numpy as jnp
from jax.experimental import pallas as pl


def kernel(emb, edge_index, d, scale):
    raise NotImplementedError("write your pallas kernel here")


# speedup vs baseline: 2.1852x; 2.1852x over previous
import functools

import jax
import jax.numpy as jnp
from jax import lax
from jax.experimental import pallas as pl
from jax.experimental.pallas import tpu as pltpu
from jax.experimental.pallas import tpu_sc as plsc

L = 16
H = 128
HC = H // L
CHUNK = 128
NCHUNK = 4
EPW = CHUNK * NCHUNK
NC = 2
NS = 16
NW = NC * NS
N_EDGE = NW * EPW


def _rsqrt(x):
    i = lax.bitcast_convert_type(x, jnp.int32)
    i = jnp.int32(0x5F3759DF) - (i >> 1)
    y = lax.bitcast_convert_type(i, jnp.float32)
    for _ in range(3):
        y = y * (jnp.float32(1.5) - jnp.float32(0.5) * x * y * y)
    return y


_GATHER_DNUMS = lax.GatherDimensionNumbers(
    offset_dims=(), collapsed_slice_dims=(0,), start_index_map=(0,))


def _shuffle(x, idx):
    return lax.gather(x, idx[:, None], _GATHER_DNUMS, slice_sizes=(1,),
                      mode=lax.GatherScatterMode.PROMISE_IN_BOUNDS)


def _lane_sum(x, lane_iota):
    for s in (1, 2, 4, 8):
        x = x + _shuffle(x, lane_iota ^ s)
    return x


def _edge_kernel(emb, uidx, vidx, dvec, scale16, out,
                 idxu, idxv, rows_u, rows_v, d_v, scale_v, out_v,
                 su0, su1, sv0, sv1):
    wid = lax.axis_index("s") * NC + lax.axis_index("c")
    pltpu.sync_copy(uidx.at[wid], idxu)
    pltpu.sync_copy(vidx.at[wid], idxv)
    pltpu.sync_copy(dvec, d_v)
    pltpu.sync_copy(scale16, scale_v)
    d_regs = [d_v[pl.ds(h * L, L)] for h in range(HC)]
    s_reg = scale_v[...]
    sems_u = (su0, su1)
    sems_v = (sv0, sv1)

    def fire(j):
        slot = j % 2
        hu = pltpu.async_copy(emb.at[idxu.at[j]], rows_u.at[slot], sems_u[slot])
        hv = pltpu.async_copy(emb.at[idxv.at[j]], rows_v.at[slot], sems_v[slot])
        return hu, hv

    handles = [fire(0)]
    for j in range(NCHUNK):
        if j + 1 < NCHUNK:
            handles.append(fire(j + 1))
        hu, hv = handles[j]
        hu.wait()
        hv.wait()
        slot = j % 2
        ru = rows_u.at[slot]
        rv = rows_v.at[slot]

        def edge_body(e, r_vec, ru=ru, rv=rv, j=j):
            dot = jnp.zeros((L,), jnp.float32)
            nu = jnp.zeros((L,), jnp.float32)
            nv = jnp.zeros((L,), jnp.float32)
            for h in range(HC):
                u = ru[e, pl.ds(h * L, L)]
                v = rv[e, pl.ds(h * L, L)]
                dot = dot + u * d_regs[h] * v
                nu = nu + u * u
                nv = nv + v * v
            ds_ = _lane_sum(dot, lane_iota)
            nus = _lane_sum(nu, lane_iota)
            nvs = _lane_sum(nv, lane_iota)
            p = jnp.maximum(nus, jnp.float32(1e-24)) * jnp.maximum(nvs, jnp.float32(1e-24))
            r = ds_ * _rsqrt(p) * s_reg
            lane = e & (L - 1)
            r_vec = jnp.where(lane_iota == lane, r, r_vec)

            @pl.when(lane == L - 1)
            def _flush(r_vec=r_vec):
                out_v[pl.ds(j * CHUNK + e - (L - 1), L)] = r_vec

            return r_vec

        lane_iota = lax.iota(jnp.int32, L)
        lax.fori_loop(0, CHUNK, edge_body, jnp.zeros((L,), jnp.float32))

    pltpu.sync_copy(out_v, out.at[wid])


_launch = functools.partial(
    pl.kernel,
    mesh=plsc.VectorSubcoreMesh(core_axis_name="c", subcore_axis_name="s"),
    out_type=jax.ShapeDtypeStruct((NW, EPW), jnp.float32),
    scratch_types=[
        pltpu.VMEM((NCHUNK, CHUNK), jnp.int32),
        pltpu.VMEM((NCHUNK, CHUNK), jnp.int32),
        pltpu.VMEM((2, CHUNK, H), jnp.float32),
        pltpu.VMEM((2, CHUNK, H), jnp.float32),
        pltpu.VMEM((H,), jnp.float32),
        pltpu.VMEM((L,), jnp.float32),
        pltpu.VMEM((EPW,), jnp.float32),
        pltpu.SemaphoreType.DMA,
        pltpu.SemaphoreType.DMA,
        pltpu.SemaphoreType.DMA,
        pltpu.SemaphoreType.DMA,
    ],
)(_edge_kernel)


def kernel(emb, edge_index, d, scale):
    assert emb.shape == (100000, H) and edge_index.shape == (2, N_EDGE)
    ei32 = edge_index.astype(jnp.int32)
    uidx = ei32[0].reshape(NW, NCHUNK, CHUNK)
    vidx = ei32[1].reshape(NW, NCHUNK, CHUNK)
    scale16 = jnp.broadcast_to(scale.astype(jnp.float32), (L,))
    out = _launch(emb, uidx, vidx, d.astype(jnp.float32), scale16)
    return out.reshape(N_EDGE)


# speedup vs baseline: 2.8434x; 1.3012x over previous
import functools

import jax
import jax.numpy as jnp
from jax import lax
from jax.experimental import pallas as pl
from jax.experimental.pallas import tpu as pltpu
from jax.experimental.pallas import tpu_sc as plsc

L = 16
H = 128
HC = H // L
CHUNK = 128
NCHUNK = 4
EPW = CHUNK * NCHUNK
NC = 2
NS = 16
NW = NC * NS
N_EDGE = NW * EPW


def _rsqrt(x):
    i = lax.bitcast_convert_type(x, jnp.int32)
    i = jnp.int32(0x5F3759DF) - (i >> 1)
    y = lax.bitcast_convert_type(i, jnp.float32)
    for _ in range(3):
        y = y * (jnp.float32(1.5) - jnp.float32(0.5) * x * y * y)
    return y


_GATHER_DNUMS = lax.GatherDimensionNumbers(
    offset_dims=(), collapsed_slice_dims=(0,), start_index_map=(0,))


def _shuffle(x, idx):
    return lax.gather(x, idx[:, None], _GATHER_DNUMS, slice_sizes=(1,),
                      mode=lax.GatherScatterMode.PROMISE_IN_BOUNDS)


def _merge(a, b, s, lane_iota):
    aa = a + _shuffle(a, lane_iota ^ s)
    bb = b + _shuffle(b, lane_iota ^ s)
    return jnp.where((lane_iota & s) == 0, aa, bb)


def _push(stack, vec, lane_iota):
    item = (0, vec)
    while stack and stack[-1][0] == item[0]:
        lv, a = stack.pop()
        item = (lv + 1, _merge(a, item[1], 1 << lv, lane_iota))
    stack.append(item)


def _edge_kernel(emb, uidx, vidx, dvec, scale16, out,
                 idxu, idxv, rows_u, rows_v, d_v, scale_v, out_v,
                 su0, su1, sv0, sv1):
    wid = lax.axis_index("s") * NC + lax.axis_index("c")
    pltpu.sync_copy(uidx.at[wid], idxu)
    pltpu.sync_copy(vidx.at[wid], idxv)
    pltpu.sync_copy(dvec, d_v)
    pltpu.sync_copy(scale16, scale_v)
    d_regs = [d_v[pl.ds(h * L, L)] for h in range(HC)]
    s_reg = scale_v[...]
    sems_u = (su0, su1)
    sems_v = (sv0, sv1)

    def fire(j):
        slot = j % 2
        hu = pltpu.async_copy(emb.at[idxu.at[j]], rows_u.at[slot], sems_u[slot])
        hv = pltpu.async_copy(emb.at[idxv.at[j]], rows_v.at[slot], sems_v[slot])
        return hu, hv

    handles = [fire(0)]
    for j in range(NCHUNK):
        if j + 1 < NCHUNK:
            handles.append(fire(j + 1))
        hu, hv = handles[j]
        hu.wait()
        hv.wait()
        slot = j % 2
        ru = rows_u.at[slot]
        rv = rows_v.at[slot]

        def group_body(g, carry, ru=ru, rv=rv, j=j):
            sd, su, sv = [], [], []
            for ei in range(L):
                e = g * L + ei
                dot = jnp.zeros((L,), jnp.float32)
                nu = jnp.zeros((L,), jnp.float32)
                nv = jnp.zeros((L,), jnp.float32)
                for h in range(HC):
                    u = ru[e, pl.ds(h * L, L)]
                    v = rv[e, pl.ds(h * L, L)]
                    dot = dot + u * d_regs[h] * v
                    nu = nu + u * u
                    nv = nv + v * v
                _push(sd, dot, lane_iota)
                _push(su, nu, lane_iota)
                _push(sv, nv, lane_iota)
            dtot, nut, nvt = sd[0][1], su[0][1], sv[0][1]
            p = jnp.maximum(nut, jnp.float32(1e-24)) * jnp.maximum(nvt, jnp.float32(1e-24))
            out_v[pl.ds(j * CHUNK + g * L, L)] = dtot * _rsqrt(p) * s_reg
            return carry

        lane_iota = lax.iota(jnp.int32, L)
        lax.fori_loop(0, CHUNK // L, group_body, 0)

    pltpu.sync_copy(out_v, out.at[wid])


_launch = functools.partial(
    pl.kernel,
    mesh=plsc.VectorSubcoreMesh(core_axis_name="c", subcore_axis_name="s"),
    out_type=jax.ShapeDtypeStruct((NW, EPW), jnp.float32),
    scratch_types=[
        pltpu.VMEM((NCHUNK, CHUNK), jnp.int32),
        pltpu.VMEM((NCHUNK, CHUNK), jnp.int32),
        pltpu.VMEM((2, CHUNK, H), jnp.float32),
        pltpu.VMEM((2, CHUNK, H), jnp.float32),
        pltpu.VMEM((H,), jnp.float32),
        pltpu.VMEM((L,), jnp.float32),
        pltpu.VMEM((EPW,), jnp.float32),
        pltpu.SemaphoreType.DMA,
        pltpu.SemaphoreType.DMA,
        pltpu.SemaphoreType.DMA,
        pltpu.SemaphoreType.DMA,
    ],
)(_edge_kernel)


def kernel(emb, edge_index, d, scale):
    assert emb.shape == (100000, H) and edge_index.shape == (2, N_EDGE)
    ei32 = edge_index.astype(jnp.int32)
    uidx = ei32[0].reshape(NW, NCHUNK, CHUNK)
    vidx = ei32[1].reshape(NW, NCHUNK, CHUNK)
    scale16 = jnp.broadcast_to(scale.astype(jnp.float32), (L,))
    out = _launch(emb, uidx, vidx, d.astype(jnp.float32), scale16)
    return out.reshape(N_EDGE)
